# baseline (speedup 1.0000x reference)
import jax
import jax.numpy as jnp
from jax.experimental import pallas as pl
from jax.experimental.pallas import tpu as pltpu

HIDDEN = 2048
RET_DIM = 512
KV_HEADS = 8
HEAD_DIM = RET_DIM // KV_HEADS
CHUNK_SIZE = 64
TOPK = 16
NUM_CHUNKS = 64
EPS = 1e-6
ROW_BLOCK = 1024
NEG_INF = float("-inf")


def _body(x_ref, wt_ref, lmr_ref, pnw_ref, ow_ref, oi_ref, *, nblk):
    rb = x_ref.shape[1]
    blk_id = pl.program_id(1)
    row0 = blk_id * rb

    x = x_ref[0]
    var = jnp.mean(x * x, axis=-1, keepdims=True)
    xn = (x * jax.lax.rsqrt(var + EPS)) * pnw_ref[0][None, :]
    q = jax.lax.dot_general(
        xn, wt_ref[...], (((1,), (0,)), ((), ())),
        preferred_element_type=jnp.float32,
        precision=jax.lax.Precision.DEFAULT,
    )

    c_row = row0 + jax.lax.broadcasted_iota(jnp.int32, (1, rb), 1)

    def _proc(h, deff, need_topk):
        d_col = jax.lax.broadcasted_iota(jnp.int32, (deff, 1), 0)
        visible = c_row >= (d_col + 1) * CHUNK_SIZE
        di = jax.lax.broadcasted_iota(jnp.int32, (deff, deff), 0)
        dj = jax.lax.broadcasted_iota(jnp.int32, (deff, deff), 1)
        gt_t = (dj > di).astype(jnp.float32)
        ge_t = (dj >= di).astype(jnp.float32)

        qh = q[:, h * HEAD_DIM:(h + 1) * HEAD_DIM]
        lmh = lmr_ref[0, h, :deff, :]
        st = jax.lax.dot_general(
            lmh, qh, (((1,), (1,)), ((), ())),
            preferred_element_type=jnp.float32,
            precision=jax.lax.Precision.DEFAULT,
        ) * 0.125
        s = jnp.where(visible, st, NEG_INF)

        if need_topk:
            cur = s
            for _ in range(TOPK):
                m = jnp.max(cur, axis=0, keepdims=True)
                cur = jnp.where(cur == m, NEG_INF, cur)
            sel = (cur == NEG_INF) & visible
        else:
            sel = visible
        sel_f = sel.astype(jnp.float32)

        p = jax.lax.dot_general(
            gt_t, sel_f, (((1,), (0,)), ((), ())),
            preferred_element_type=jnp.float32,
            precision=jax.lax.Precision.DEFAULT,
        )
        sp = jnp.where(s > 15.0, s, jnp.log1p(jnp.exp(jnp.minimum(s, 15.0))))
        spm = jnp.where(sel, sp, 0.0)
        rc = jax.lax.dot_general(
            ge_t, spm, (((1,), (0,)), ((), ())),
            preferred_element_type=jnp.float32,
            precision=jax.lax.Precision.HIGHEST,
        )
        w64 = jnp.where(sel, jnp.exp(s - rc), 0.0)

        p_sel = jnp.where(sel, p.astype(jnp.int32), TOPK)
        packed = d_col.astype(jnp.float32) + w64 * 0.5
        j3 = jax.lax.broadcasted_iota(jnp.int32, (TOPK, deff, rb), 0)
        oh = p_sel[None] == j3
        return jnp.sum(jnp.where(oh, packed[None], 0.0), axis=1)

    def _dispatch(h, lo, hi):
        if hi - lo == 1:
            deff = min(NUM_CHUNKS, (lo + 1) * rb // CHUNK_SIZE)
            need_topk = (lo + 1) * rb > (TOPK + 1) * CHUNK_SIZE
            return lambda: _proc(h, deff, need_topk)
        mid = (lo + hi) // 2
        return lambda: jax.lax.cond(
            blk_id < mid, _dispatch(h, lo, mid), _dispatch(h, mid, hi))

    for h in range(KV_HEADS):
        opk = _dispatch(h, 0, nblk)()
        oi = jnp.floor(opk)
        ow_ref[0, h] = (opk - oi) * 2.0
        oi_ref[0, h] = oi.astype(jnp.int32)


def kernel(hidden_states, landmarks, pre_norm_weight, ln_weight):
    n, seq, _ = hidden_states.shape
    wt = ln_weight.T
    lmr = jnp.transpose(landmarks, (0, 2, 1, 3))
    pnw = pre_norm_weight.reshape(1, HIDDEN)

    import functools
    grid = (n, seq // ROW_BLOCK)
    ow, oi = pl.pallas_call(
        functools.partial(_body, nblk=seq // ROW_BLOCK),
        grid=grid,
        in_specs=[
            pl.BlockSpec((1, ROW_BLOCK, HIDDEN), lambda b, i: (b, i, 0)),
            pl.BlockSpec((HIDDEN, RET_DIM), lambda b, i: (0, 0)),
            pl.BlockSpec((1, KV_HEADS, NUM_CHUNKS, HEAD_DIM), lambda b, i: (b, 0, 0, 0)),
            pl.BlockSpec((1, HIDDEN), lambda b, i: (0, 0)),
        ],
        out_specs=[
            pl.BlockSpec((1, KV_HEADS, TOPK, ROW_BLOCK), lambda b, i: (b, 0, 0, i)),
            pl.BlockSpec((1, KV_HEADS, TOPK, ROW_BLOCK), lambda b, i: (b, 0, 0, i)),
        ],
        out_shape=[
            jax.ShapeDtypeStruct((n, KV_HEADS, TOPK, seq), jnp.float32),
            jax.ShapeDtypeStruct((n, KV_HEADS, TOPK, seq), jnp.int32),
        ],
        compiler_params=pltpu.CompilerParams(
            dimension_semantics=("parallel", "parallel"),
        ),
    )(hidden_states, wt, lmr, pnw)

    chunk_weights = jnp.transpose(ow, (0, 3, 1, 2))
    idx_final = jnp.transpose(oi, (0, 3, 1, 2))
    return hidden_states, chunk_weights, landmarks, idx_final


# speedup vs baseline: 1.8909x; 1.8909x over previous
import functools

import jax
import jax.numpy as jnp
from jax.experimental import pallas as pl
from jax.experimental.pallas import tpu as pltpu

HIDDEN = 2048
RET_DIM = 512
KV_HEADS = 8
HEAD_DIM = RET_DIM // KV_HEADS
CHUNK_SIZE = 64
TOPK = 16
NUM_CHUNKS = 64
EPS = 1e-6
ROW_BLOCK = 1024
NEG_INF = float("-inf")


def _body(x_ref, wt_ref, lmr_ref, pnw_ref, ow_ref, oi_ref, *,
          row0, deff, need_topk):
    rb = x_ref.shape[1]

    x = x_ref[0]
    var = jnp.mean(x * x, axis=-1, keepdims=True)
    xn = (x * jax.lax.rsqrt(var + EPS)) * pnw_ref[0][None, :]
    q = jax.lax.dot_general(
        xn, wt_ref[...], (((1,), (0,)), ((), ())),
        preferred_element_type=jnp.float32,
        precision=jax.lax.Precision.DEFAULT,
    )

    c_row = row0 + jax.lax.broadcasted_iota(jnp.int32, (1, rb), 1)
    d_col = jax.lax.broadcasted_iota(jnp.int32, (deff, 1), 0)
    visible = c_row >= (d_col + 1) * CHUNK_SIZE
    di = jax.lax.broadcasted_iota(jnp.int32, (deff, deff), 0)
    dj = jax.lax.broadcasted_iota(jnp.int32, (deff, deff), 1)
    gt_t = (dj > di).astype(jnp.float32)
    ge_t = (dj >= di).astype(jnp.float32)

    for h in range(KV_HEADS):
        qh = q[:, h * HEAD_DIM:(h + 1) * HEAD_DIM]
        lmh = lmr_ref[0, h, :deff, :]
        st = jax.lax.dot_general(
            lmh, qh, (((1,), (1,)), ((), ())),
            preferred_element_type=jnp.float32,
            precision=jax.lax.Precision.DEFAULT,
        ) * 0.125
        s = jnp.where(visible, st, NEG_INF)

        if need_topk:
            cur = s
            for _ in range(TOPK):
                m = jnp.max(cur, axis=0, keepdims=True)
                cur = jnp.where(cur == m, NEG_INF, cur)
            sel = (cur == NEG_INF) & visible
        else:
            sel = visible
        sel_f = sel.astype(jnp.float32)

        p = jax.lax.dot_general(
            gt_t, sel_f, (((1,), (0,)), ((), ())),
            preferred_element_type=jnp.float32,
            precision=jax.lax.Precision.DEFAULT,
        )
        sp = jnp.where(s > 15.0, s, jnp.log1p(jnp.exp(jnp.minimum(s, 15.0))))
        spm = jnp.where(sel, sp, 0.0)
        rc = jax.lax.dot_general(
            ge_t, spm, (((1,), (0,)), ((), ())),
            preferred_element_type=jnp.float32,
            precision=jax.lax.Precision.HIGHEST,
        )
        w64 = jnp.where(sel, jnp.exp(s - rc), 0.0)

        p_sel = jnp.where(sel, p.astype(jnp.int32), TOPK)
        packed = d_col.astype(jnp.float32) + w64 * 0.5
        j3 = jax.lax.broadcasted_iota(jnp.int32, (TOPK, deff, rb), 0)
        oh = p_sel[None] == j3
        opk = jnp.sum(jnp.where(oh, packed[None], 0.0), axis=1)
        oi = jnp.floor(opk)
        ow_ref[0, h] = (opk - oi) * 2.0
        oi_ref[0, h] = oi.astype(jnp.int32)


def kernel(hidden_states, landmarks, pre_norm_weight, ln_weight):
    n, seq, _ = hidden_states.shape
    wt = ln_weight.T
    lmr = jnp.transpose(landmarks, (0, 2, 1, 3))
    pnw = pre_norm_weight.reshape(1, HIDDEN)

    nblk = seq // ROW_BLOCK
    ows, ois = [], []
    for blk in range(nblk):
        deff = min(NUM_CHUNKS, (blk + 1) * ROW_BLOCK // CHUNK_SIZE)
        need_topk = (blk + 1) * ROW_BLOCK > (TOPK + 1) * CHUNK_SIZE
        ow, oi = pl.pallas_call(
            functools.partial(_body, row0=blk * ROW_BLOCK, deff=deff,
                              need_topk=need_topk),
            grid=(n,),
            in_specs=[
                pl.BlockSpec((1, ROW_BLOCK, HIDDEN),
                             lambda b, _blk=blk: (b, _blk, 0)),
                pl.BlockSpec((HIDDEN, RET_DIM), lambda b: (0, 0)),
                pl.BlockSpec((1, KV_HEADS, NUM_CHUNKS, HEAD_DIM),
                             lambda b: (b, 0, 0, 0)),
                pl.BlockSpec((1, HIDDEN), lambda b: (0, 0)),
            ],
            out_specs=[
                pl.BlockSpec((1, KV_HEADS, TOPK, ROW_BLOCK),
                             lambda b: (b, 0, 0, 0)),
                pl.BlockSpec((1, KV_HEADS, TOPK, ROW_BLOCK),
                             lambda b: (b, 0, 0, 0)),
            ],
            out_shape=[
                jax.ShapeDtypeStruct((n, KV_HEADS, TOPK, ROW_BLOCK), jnp.float32),
                jax.ShapeDtypeStruct((n, KV_HEADS, TOPK, ROW_BLOCK), jnp.int32),
            ],
            compiler_params=pltpu.CompilerParams(
                dimension_semantics=("parallel",),
            ),
        )(hidden_states, wt, lmr, pnw)
        ows.append(ow)
        ois.append(oi)

    ow_full = jnp.concatenate(ows, axis=3)
    oi_full = jnp.concatenate(ois, axis=3)
    chunk_weights = jnp.transpose(ow_full, (0, 3, 1, 2))
    idx_final = jnp.transpose(oi_full, (0, 3, 1, 2))
    return hidden_states, chunk_weights, landmarks, idx_final
